# Initial kernel scaffold; baseline (speedup 1.0000x reference)
import jax
import jax.numpy as jnp
from jax.experimental import pallas as pl


def kernel(I, T, L, S, B, dims, temp):
    raise NotImplementedError("write your pallas kernel here")


# speedup vs baseline: 1.3901x; 1.3901x over previous
import math
import jax
import jax.numpy as jnp
from jax.experimental import pallas as pl

_C, _K, _D_IN, _M, _N = 64, 16, 1024, 1024, 8192
_NODES = _K - 1
_EPS = 1e-8

_N_TILE = 512


def _tc_body(invt_ref, i_ref, sft_ref, tf_ref, wp_ref, wm_ref, lf_ref, o_ref):
    invt = invt_ref[0, 0]
    h = jnp.dot(i_ref[...], sft_ref[...], preferred_element_type=jnp.float32)
    z = (h - tf_ref[0, :][None, :]) * invt
    sig = jax.nn.sigmoid(z)
    logp = jnp.log(sig + _EPS)
    logm = jnp.log(1.0 - sig + _EPS)
    log_e = (
        jnp.dot(logp, wp_ref[...], preferred_element_type=jnp.float32)
        + jnp.dot(logm, wm_ref[...], preferred_element_type=jnp.float32)
    )
    e = jnp.exp(log_e)
    o_ref[...] = jnp.dot(e, lf_ref[...], preferred_element_type=jnp.float32)


def kernel(I, T, L, S, B, dims, temp):
    n = I.shape[0]
    sft = S.reshape(_C * _NODES, _D_IN).T
    t_flat = T.reshape(1, _C * _NODES)
    inv_t = (1.0 / temp[0]).reshape(1, 1)
    bp = (B > 0.5).astype(jnp.float32)
    bm = (B < -0.5).astype(jnp.float32)
    eye = jnp.eye(_C, dtype=jnp.float32)
    wp = jnp.einsum("cd,kj->cjdk", eye, bp).reshape(_C * _NODES, _C * _K)
    wm = jnp.einsum("cd,kj->cjdk", eye, bm).reshape(_C * _NODES, _C * _K)
    lf = L.reshape(_M, _C * _K).T

    grid = (n // _N_TILE,)
    out = pl.pallas_call(
        _tc_body,
        grid=grid,
        in_specs=[
            pl.BlockSpec((1, 1), lambda i: (0, 0)),
            pl.BlockSpec((_N_TILE, _D_IN), lambda i: (i, 0)),
            pl.BlockSpec((_D_IN, _C * _NODES), lambda i: (0, 0)),
            pl.BlockSpec((1, _C * _NODES), lambda i: (0, 0)),
            pl.BlockSpec((_C * _NODES, _C * _K), lambda i: (0, 0)),
            pl.BlockSpec((_C * _NODES, _C * _K), lambda i: (0, 0)),
            pl.BlockSpec((_C * _K, _M), lambda i: (0, 0)),
        ],
        out_specs=pl.BlockSpec((_N_TILE, _M), lambda i: (i, 0)),
        out_shape=jax.ShapeDtypeStruct((n, _M), jnp.float32),
    )(inv_t, I, sft, t_flat, wp, wm, lf)
    return out
